# baseline (speedup 1.0000x reference)
import jax
import jax.numpy as jnp
from jax import lax
from jax.experimental import pallas as pl
from jax.experimental.pallas import tpu as pltpu
from jax.experimental.pallas import tpu_sc as plsc

NUM_SEG = 16
TOTAL_TOK = 32768
D = 128

NC = 2
NS = 16
NW = NC * NS

T_SC = 8192
T_TC = TOTAL_TOK - T_SC
TW = T_SC // NW
R = 128
NCHUNK = TW // R

TC_BLK = 8192
NSTEP = T_TC // TC_BLK
CNT_BLK = (-(-TOTAL_TOK // NSTEP) + D - 1) // D * D
PAD_TOK = NSTEP * CNT_BLK


def _sc_body(flat_hbm, sids_hbm, zacc_hbm, psums_hbm,
             sid_v, rows_a, rows_b, shared_acc, sem_a, sem_b, sem_s):
    c = lax.axis_index("c")
    s = lax.axis_index("s")
    wid = c * NS + s
    base = pl.multiple_of(wid * TW, R)

    bufs = (rows_a, rows_b)
    sems = (sem_a, sem_b)

    row0 = wid * NCHUNK
    win0 = pl.multiple_of(row0 // 8 * 8, 8)
    pltpu.sync_copy(sids_hbm.at[pl.ds(win0, 16)], sid_v)
    roff = row0 - win0

    @pl.when(s == 0)
    def _init():
        pltpu.sync_copy(zacc_hbm, shared_acc)

    plsc.subcore_barrier()

    pending = pltpu.async_copy(flat_hbm.at[pl.ds(base, R)], bufs[0], sems[0])
    scatters = []
    for k in range(NCHUNK):
        nxt = None
        if k + 1 < NCHUNK:
            nxt = pltpu.async_copy(
                flat_hbm.at[pl.ds(base + (k + 1) * R, R)],
                bufs[(k + 1) % 2], sems[(k + 1) % 2])
        pending.wait()
        scatters.append(pltpu.async_copy(
            bufs[k % 2], shared_acc.at[sid_v.at[roff + k]], sem_s, add=True))
        pending = nxt

    for h in scatters:
        h.wait()

    plsc.subcore_barrier()

    @pl.when(s == 0)
    def _fini():
        pltpu.sync_copy(shared_acc, psums_hbm.at[c])


def _tc_psum_body(rows_ref, oht_ref, ohtc_ref, o_ref, cnt_ref):
    part = lax.dot_general(oht_ref[...].astype(jnp.float32), rows_ref[...],
                           (((1,), (0,)), ((), ())),
                           preferred_element_type=jnp.float32)
    cpart = jnp.sum(ohtc_ref[...].astype(jnp.float32), axis=1,
                    keepdims=True)

    @pl.when(pl.program_id(0) == 0)
    def _():
        o_ref[...] = jnp.zeros_like(o_ref)
        cnt_ref[...] = jnp.zeros_like(cnt_ref)

    o_ref[...] += part
    cnt_ref[...] += cpart


def _combine_body(ps_sc_ref, ps_tc_ref, cnt_ref, o_ref):
    cnt = jnp.maximum(cnt_ref[...], 1.0)
    o_ref[...] = (ps_sc_ref[0] + ps_sc_ref[1] + ps_tc_ref[...]) / cnt


def kernel(flat, segment_ids):
    sids = segment_ids.astype(jnp.int32)
    sids2 = sids.reshape(TOTAL_TOK // R, R)
    zacc = jnp.zeros((NUM_SEG, D), jnp.float32)

    mesh = plsc.VectorSubcoreMesh(core_axis_name="c", subcore_axis_name="s")
    psums_sc = pl.kernel(
        _sc_body,
        out_type=jax.ShapeDtypeStruct((NC, NUM_SEG, D), jnp.float32),
        mesh=mesh,
        scratch_types=[
            pltpu.VMEM((16, R), jnp.int32),
            pltpu.VMEM((R, D), jnp.float32),
            pltpu.VMEM((R, D), jnp.float32),
            pltpu.VMEM_SHARED((NUM_SEG, D), jnp.float32),
            pltpu.SemaphoreType.DMA,
            pltpu.SemaphoreType.DMA,
            pltpu.SemaphoreType.DMA,
        ],
    )(flat, sids2, zacc)

    sids_pad = jnp.concatenate(
        [sids, jnp.full((PAD_TOK - TOTAL_TOK,), -1, jnp.int32)])
    oht = (sids_pad[None, :] == lax.broadcasted_iota(
        jnp.int32, (NUM_SEG, PAD_TOK), 0)).astype(jnp.bfloat16)
    off = T_SC // TC_BLK
    psum_tc, cnt = pl.pallas_call(
        _tc_psum_body,
        grid=(NSTEP,),
        in_specs=[
            pl.BlockSpec((TC_BLK, D), lambda k: (k + off, 0)),
            pl.BlockSpec((NUM_SEG, TC_BLK), lambda k: (0, k + off)),
            pl.BlockSpec((NUM_SEG, CNT_BLK), lambda k: (0, k)),
        ],
        out_specs=[
            pl.BlockSpec((NUM_SEG, D), lambda k: (0, 0)),
            pl.BlockSpec((NUM_SEG, 1), lambda k: (0, 0)),
        ],
        out_shape=[
            jax.ShapeDtypeStruct((NUM_SEG, D), jnp.float32),
            jax.ShapeDtypeStruct((NUM_SEG, 1), jnp.float32),
        ],
    )(flat, oht, oht)

    out = pl.pallas_call(
        _combine_body,
        out_shape=jax.ShapeDtypeStruct((NUM_SEG, D), jnp.float32),
    )(psums_sc, psum_tc, cnt)
    return out


# speedup vs baseline: 1.0288x; 1.0288x over previous
import jax
import jax.numpy as jnp
from jax import lax
from jax.experimental import pallas as pl
from jax.experimental.pallas import tpu as pltpu
from jax.experimental.pallas import tpu_sc as plsc

NUM_SEG = 16
TOTAL_TOK = 32768
D = 128

NC = 2
NS = 16
NW = NC * NS

T_SC = 8192
T_TC = TOTAL_TOK - T_SC
TW = T_SC // NW
R = 128
NCHUNK = TW // R

TC_BLK = 4096
NSTEP = T_TC // TC_BLK
CNT_BLK = (-(-TOTAL_TOK // NSTEP) + D - 1) // D * D
PAD_TOK = NSTEP * CNT_BLK


def _sc_body(flat_hbm, sids_hbm, zacc_hbm, psums_hbm,
             sid_v, rows_a, rows_b, shared_acc, sem_a, sem_b, sem_s):
    c = lax.axis_index("c")
    s = lax.axis_index("s")
    wid = c * NS + s
    base = pl.multiple_of(wid * TW, R)

    bufs = (rows_a, rows_b)
    sems = (sem_a, sem_b)

    row0 = wid * NCHUNK
    win0 = pl.multiple_of(row0 // 8 * 8, 8)
    pltpu.sync_copy(sids_hbm.at[pl.ds(win0, 16)], sid_v)
    roff = row0 - win0

    @pl.when(s == 0)
    def _init():
        pltpu.sync_copy(zacc_hbm, shared_acc)

    plsc.subcore_barrier()

    pending = pltpu.async_copy(flat_hbm.at[pl.ds(base, R)], bufs[0], sems[0])
    scatters = []
    for k in range(NCHUNK):
        nxt = None
        if k + 1 < NCHUNK:
            nxt = pltpu.async_copy(
                flat_hbm.at[pl.ds(base + (k + 1) * R, R)],
                bufs[(k + 1) % 2], sems[(k + 1) % 2])
        pending.wait()
        scatters.append(pltpu.async_copy(
            bufs[k % 2], shared_acc.at[sid_v.at[roff + k]], sem_s, add=True))
        pending = nxt

    for h in scatters:
        h.wait()

    plsc.subcore_barrier()

    @pl.when(s == 0)
    def _fini():
        pltpu.sync_copy(shared_acc, psums_hbm.at[c])


def _tc_psum_body(rows_ref, oht_ref, ohtc_ref, o_ref, cnt_ref):
    part = lax.dot_general(oht_ref[...].astype(jnp.float32), rows_ref[...],
                           (((1,), (0,)), ((), ())),
                           preferred_element_type=jnp.float32)
    cpart = jnp.sum(ohtc_ref[...].astype(jnp.float32), axis=1,
                    keepdims=True)

    @pl.when(pl.program_id(0) == 0)
    def _():
        o_ref[...] = jnp.zeros_like(o_ref)
        cnt_ref[...] = jnp.zeros_like(cnt_ref)

    o_ref[...] += part
    cnt_ref[...] += cpart


def _combine_body(ps_sc_ref, ps_tc_ref, cnt_ref, o_ref):
    cnt = jnp.maximum(cnt_ref[...], 1.0)
    o_ref[...] = (ps_sc_ref[0] + ps_sc_ref[1] + ps_tc_ref[...]) / cnt


def kernel(flat, segment_ids):
    sids = segment_ids.astype(jnp.int32)
    sids2 = sids.reshape(TOTAL_TOK // R, R)
    zacc = jnp.zeros((NUM_SEG, D), jnp.float32)

    mesh = plsc.VectorSubcoreMesh(core_axis_name="c", subcore_axis_name="s")
    psums_sc = pl.kernel(
        _sc_body,
        out_type=jax.ShapeDtypeStruct((NC, NUM_SEG, D), jnp.float32),
        mesh=mesh,
        scratch_types=[
            pltpu.VMEM((16, R), jnp.int32),
            pltpu.VMEM((R, D), jnp.float32),
            pltpu.VMEM((R, D), jnp.float32),
            pltpu.VMEM_SHARED((NUM_SEG, D), jnp.float32),
            pltpu.SemaphoreType.DMA,
            pltpu.SemaphoreType.DMA,
            pltpu.SemaphoreType.DMA,
        ],
    )(flat, sids2, zacc)

    sids_pad = jnp.concatenate(
        [sids, jnp.full((PAD_TOK - TOTAL_TOK,), -1, jnp.int32)])
    oht = (sids_pad[None, :] == lax.broadcasted_iota(
        jnp.int32, (NUM_SEG, PAD_TOK), 0)).astype(jnp.bfloat16)
    off = T_SC // TC_BLK
    psum_tc, cnt = pl.pallas_call(
        _tc_psum_body,
        grid=(NSTEP,),
        in_specs=[
            pl.BlockSpec((TC_BLK, D), lambda k: (k + off, 0)),
            pl.BlockSpec((NUM_SEG, TC_BLK), lambda k: (0, k + off)),
            pl.BlockSpec((NUM_SEG, CNT_BLK), lambda k: (0, k)),
        ],
        out_specs=[
            pl.BlockSpec((NUM_SEG, D), lambda k: (0, 0)),
            pl.BlockSpec((NUM_SEG, 1), lambda k: (0, 0)),
        ],
        out_shape=[
            jax.ShapeDtypeStruct((NUM_SEG, D), jnp.float32),
            jax.ShapeDtypeStruct((NUM_SEG, 1), jnp.float32),
        ],
    )(flat, oht, oht)

    out = pl.pallas_call(
        _combine_body,
        out_shape=jax.ShapeDtypeStruct((NUM_SEG, D), jnp.float32),
    )(psums_sc, psum_tc, cnt)
    return out
